# baseline (speedup 1.0000x reference)
import jax
import jax.numpy as jnp
from jax import lax
from jax.experimental import pallas as pl
from jax.experimental.pallas import tpu as pltpu
from jax.experimental.pallas import tpu_sc as plsc

EMBED_DIM = 32
PAD_DIM = 128
CHUNK = 128
NBUF = 4
NUM_CORES = 2
NUM_SUBCORES = 16
NUM_WORKERS = NUM_CORES * NUM_SUBCORES
SLICE_BLOCK = 16


def _slice_body(x_ref, o_ref):
    o_ref[...] = x_ref[:, :, :EMBED_DIM]


def kernel(input, table):
    batch, hist = input.shape
    num_indices = batch * hist
    b_per_w = num_indices // NUM_WORKERS
    nchunks = b_per_w // CHUNK
    assert nchunks % NBUF == 0 and nchunks >= 2 * NBUF
    indices = input.reshape(num_indices)
    table_pad = jnp.pad(table, ((0, 0), (0, PAD_DIM - EMBED_DIM)))

    mesh = plsc.VectorSubcoreMesh(core_axis_name="core",
                                  subcore_axis_name="subcore")

    @pl.kernel(
        out_type=jax.ShapeDtypeStruct((num_indices, PAD_DIM), jnp.float32),
        mesh=mesh,
        scratch_types=[
            pltpu.VMEM((b_per_w,), jnp.int32),
            [pltpu.VMEM((CHUNK, PAD_DIM), jnp.float32)
             for _ in range(NBUF)],
            [pltpu.SemaphoreType.DMA for _ in range(NBUF)],
            [pltpu.SemaphoreType.DMA for _ in range(NBUF)],
        ],
    )
    def gather_kernel(tab_hbm, idx_hbm, out_hbm, idx_v, rows, gsems, wsems):
        wid = lax.axis_index("subcore") * NUM_CORES + lax.axis_index("core")
        base = wid * b_per_w
        pltpu.sync_copy(idx_hbm.at[pl.ds(base, b_per_w)], idx_v)

        def gather(chunk, b):
            pltpu.async_copy(
                tab_hbm.at[idx_v.at[pl.ds(chunk * CHUNK, CHUNK)]], rows[b],
                gsems[b])

        def wait_gather(chunk, b):
            pltpu.make_async_copy(
                tab_hbm.at[idx_v.at[pl.ds(chunk * CHUNK, CHUNK)]], rows[b],
                gsems[b]).wait()

        def out_slice(chunk):
            return out_hbm.at[pl.ds((base + chunk * CHUNK), CHUNK)]

        def write(chunk, b):
            pltpu.async_copy(rows[b], out_slice(chunk), wsems[b])

        def wait_write(chunk, b):
            pltpu.make_async_copy(rows[b], out_slice(chunk),
                                  wsems[b]).wait()

        for b in range(NBUF):
            gather(b, b)

        @pl.loop(0, nchunks - NBUF, step=NBUF)
        def _(k):
            for b in range(NBUF):
                c = k + b
                wait_gather(c, b)
                write(c, b)
                wait_write(c, b)
                gather(c + NBUF, b)

        for b in range(NBUF):
            c = nchunks - NBUF + b
            wait_gather(c, b)
            write(c, b)
            wait_write(c, b)

    out_pad = gather_kernel(table_pad, indices)

    out3d = out_pad.reshape(batch, hist, PAD_DIM)
    out = pl.pallas_call(
        _slice_body,
        grid=(batch // SLICE_BLOCK,),
        in_specs=[pl.BlockSpec((SLICE_BLOCK, hist, PAD_DIM),
                               lambda i: (i, 0, 0))],
        out_specs=pl.BlockSpec((SLICE_BLOCK, hist, EMBED_DIM),
                               lambda i: (i, 0, 0)),
        out_shape=jax.ShapeDtypeStruct((batch, hist, EMBED_DIM),
                                       jnp.float32),
    )(out3d)
    return out


# speedup vs baseline: 1.4523x; 1.4523x over previous
import jax
import jax.numpy as jnp
from jax import lax
from jax.experimental import pallas as pl
from jax.experimental.pallas import tpu as pltpu
from jax.experimental.pallas import tpu_sc as plsc

EMBED_DIM = 32
PAD_DIM = 128
CHUNK = 128
NBUF = 5
NUM_CORES = 2
NUM_SUBCORES = 16
NUM_WORKERS = NUM_CORES * NUM_SUBCORES


def kernel(input, table):
    batch, hist = input.shape
    num_indices = batch * hist
    b_per_w = num_indices // NUM_WORKERS
    nchunks = b_per_w // CHUNK
    assert nchunks % NBUF == 0 and nchunks >= 2 * NBUF
    indices = input.reshape(num_indices)
    table_pad = jnp.pad(table, ((0, 0), (0, PAD_DIM - EMBED_DIM)))

    mesh = plsc.VectorSubcoreMesh(core_axis_name="core",
                                  subcore_axis_name="subcore")

    @pl.kernel(
        out_type=jax.ShapeDtypeStruct((num_indices, PAD_DIM), jnp.float32),
        mesh=mesh,
        scratch_types=[
            pltpu.VMEM((b_per_w,), jnp.int32),
            [pltpu.VMEM((CHUNK, PAD_DIM), jnp.float32)
             for _ in range(NBUF)],
            [pltpu.SemaphoreType.DMA for _ in range(NBUF)],
            [pltpu.SemaphoreType.DMA for _ in range(NBUF)],
        ],
    )
    def gather_kernel(tab_hbm, idx_hbm, out_hbm, idx_v, rows, gsems, wsems):
        wid = lax.axis_index("subcore") * NUM_CORES + lax.axis_index("core")
        base = wid * b_per_w
        pltpu.sync_copy(idx_hbm.at[pl.ds(base, b_per_w)], idx_v)

        def gather(chunk, b):
            pltpu.async_copy(
                tab_hbm.at[idx_v.at[pl.ds(chunk * CHUNK, CHUNK)]], rows[b],
                gsems[b])

        def wait_gather(chunk, b):
            pltpu.make_async_copy(
                tab_hbm.at[idx_v.at[pl.ds(chunk * CHUNK, CHUNK)]], rows[b],
                gsems[b]).wait()

        def out_slice(chunk):
            return out_hbm.at[pl.ds((base + chunk * CHUNK), CHUNK)]

        def write(chunk, b):
            pltpu.async_copy(rows[b], out_slice(chunk), wsems[b])

        def wait_write(chunk, b):
            pltpu.make_async_copy(rows[b], out_slice(chunk),
                                  wsems[b]).wait()

        for b in range(NBUF):
            gather(b, b)

        @pl.loop(0, nchunks - NBUF, step=NBUF)
        def _(k):
            for b in range(NBUF):
                c = k + b
                wait_gather(c, b)
                write(c, b)
                wait_write(c, b)
                gather(c + NBUF, b)

        for b in range(NBUF):
            c = nchunks - NBUF + b
            wait_gather(c, b)
            write(c, b)
            wait_write(c, b)

    out_pad = gather_kernel(table_pad, indices)
    return out_pad[:, :EMBED_DIM].reshape(batch, hist, EMBED_DIM)
